# baseline (speedup 1.0000x reference)
import numpy as np
import jax
import jax.numpy as jnp
from jax.experimental import pallas as pl
from jax.experimental.pallas import tpu as pltpu


def _line_constants(num_rows: int):
    rng = np.random.default_rng(0)
    rand_row = int(rng.integers(0, num_rows))
    x_min, alpha = 1.0728769e-07, 1.0868737
    r = float(rng.random())
    relative_error = x_min * (1.0 - r) ** (-1.0 / (alpha - 1.0))
    return rand_row, relative_error


_BLOCK_ROWS = 4096


def kernel(forward_input):
    n_rows, n_cols = forward_input.shape
    rand_row, rel_err = _line_constants(n_rows)

    block_rows = _BLOCK_ROWS
    grid = n_rows // block_rows
    target_block = rand_row // block_rows
    row_off = rand_row % block_rows

    def body(x_ref, o_hbm, sem):
        i = pl.program_id(0)

        @pl.when(i == target_block)
        def _():
            x_ref[row_off, :] = x_ref[row_off, :] * jnp.float32(rel_err)

        cp = pltpu.make_async_copy(
            x_ref, o_hbm.at[pl.ds(i * block_rows, block_rows)], sem
        )
        cp.start()
        cp.wait()

    return pl.pallas_call(
        body,
        grid=(grid,),
        in_specs=[pl.BlockSpec((block_rows, n_cols), lambda i: (i, 0))],
        out_specs=pl.BlockSpec(memory_space=pl.ANY),
        out_shape=jax.ShapeDtypeStruct((n_rows, n_cols), forward_input.dtype),
        scratch_shapes=[pltpu.SemaphoreType.DMA],
    )(forward_input)
